# baseline (speedup 1.0000x reference)
import jax
import jax.numpy as jnp
from jax.experimental import pallas as pl


def _prep_body(inp_ref, e_ref, out_ref):
    out_ref[...] = inp_ref[...] * e_ref[...]


def _splat_one(cat, flow):
    C, H, W = cat.shape
    gx = jnp.arange(W, dtype=jnp.float32)[None, :]
    gy = jnp.arange(H, dtype=jnp.float32)[:, None]
    ox = gx + flow[0]
    oy = gy + flow[1]
    nwx = jnp.floor(ox)
    nwy = jnp.floor(oy)
    flat = cat.reshape(C, H * W)
    out = jnp.zeros((C, H * W), dtype=cat.dtype)
    for dx, dy in ((0, 0), (1, 0), (0, 1), (1, 1)):
        cx = nwx + float(dx)
        cy = nwy + float(dy)
        wx = ((nwx + 1.0) - ox) if dx == 0 else (ox - nwx)
        wy = ((nwy + 1.0) - oy) if dy == 0 else (oy - nwy)
        w = wx * wy
        valid = (cx >= 0.0) & (cx <= float(W - 1)) & (cy >= 0.0) & (cy <= float(H - 1))
        cxi = jnp.clip(cx.astype(jnp.int32), 0, W - 1)
        cyi = jnp.clip(cy.astype(jnp.int32), 0, H - 1)
        idx = (cyi * W + cxi).reshape(-1)
        wv = jnp.where(valid, w, 0.0).reshape(-1)
        out = out.at[:, idx].add(flat * wv[None, :])
    return out.reshape(C, H, W)


def kernel(tenInput, tenFlow, tenMetric):
    B, C, H, W = tenInput.shape
    e = jnp.exp(tenMetric)
    scaled = pl.pallas_call(
        _prep_body,
        out_shape=jax.ShapeDtypeStruct((B, C, H, W), jnp.float32),
        grid=(B, C // 8),
        in_specs=[
            pl.BlockSpec((1, 8, H, W), lambda b, c: (b, c, 0, 0)),
            pl.BlockSpec((1, 1, H, W), lambda b, c: (b, 0, 0, 0)),
        ],
        out_specs=pl.BlockSpec((1, 8, H, W), lambda b, c: (b, c, 0, 0)),
    )(tenInput, e)
    cat = jnp.concatenate([scaled, e], axis=1)
    splatted = jax.vmap(_splat_one)(cat, tenFlow)
    out = splatted[:, :-1] / (splatted[:, -1:] + 1e-7)
    return out


# speedup vs baseline: 2.0153x; 2.0153x over previous
import jax
import jax.numpy as jnp
from jax import lax
from jax.experimental import pallas as pl
from jax.experimental.pallas import tpu as pltpu
from jax.experimental.pallas import tpu_sc as plsc

H = 384
W = 384
HW = H * W
B = 2
C = 96
NG = 12
NS = 16
PT = HW // NS
PBA = 2048
PBB = 16384

_CORNERS = ((0, 0), (1, 0), (0, 1), (1, 1))


def _corners_body(flow_ref, met_ref, wgt_ref, idx_ref):
    j = pl.program_id(1)
    f0 = flow_ref[0, 0:1, :]
    f1 = flow_ref[0, 1:2, :]
    e = jnp.exp(met_ref[0, :, :])
    pix = j * PBA + lax.broadcasted_iota(jnp.int32, (1, PBA), 1)
    y = pix // W
    x = pix - y * W
    ox = jnp.clip(x.astype(jnp.float32) + f0, -3.0, W + 2.0)
    oy = jnp.clip(y.astype(jnp.float32) + f1, -3.0, H + 2.0)
    nwx = jnp.floor(ox)
    nwy = jnp.floor(oy)
    for ci, (dx, dy) in enumerate(_CORNERS):
        cx = nwx + float(dx)
        cy = nwy + float(dy)
        wx = ((nwx + 1.0) - ox) if dx == 0 else (ox - nwx)
        wy = ((nwy + 1.0) - oy) if dy == 0 else (oy - nwy)
        valid = ((cx >= 0.0) & (cx <= W - 1.0) & (cy >= 0.0) & (cy <= H - 1.0))
        w = jnp.where(valid, wx * wy, 0.0) * e
        cxi = jnp.clip(cx.astype(jnp.int32), 0, W - 1)
        cyi = jnp.clip(cy.astype(jnp.int32), 0, H - 1)
        wgt_ref[0, ci:ci + 1, :] = w
        idx_ref[0, ci:ci + 1, :] = cyi * W + cxi


def _corners(flow, met):
    return pl.pallas_call(
        _corners_body,
        out_shape=(jax.ShapeDtypeStruct((B, 4, HW), jnp.float32),
                   jax.ShapeDtypeStruct((B, 4, HW), jnp.int32)),
        grid=(B, HW // PBA),
        in_specs=[
            pl.BlockSpec((1, 2, PBA), lambda b, j: (b, 0, j)),
            pl.BlockSpec((1, 1, PBA), lambda b, j: (b, 0, j)),
        ],
        out_specs=(pl.BlockSpec((1, 4, PBA), lambda b, j: (b, 0, j)),
                   pl.BlockSpec((1, 4, PBA), lambda b, j: (b, 0, j))),
    )(flow, met)


def _values_body(inp_ref, wgt_ref, v_ref):
    v_ref[0, 0, 0] = inp_ref[0, 0] * wgt_ref[0, 0]


def _values(inp12, wgt):
    return pl.pallas_call(
        _values_body,
        out_shape=jax.ShapeDtypeStruct((B, NG, 4, 8, HW), jnp.float32),
        grid=(B, NG, 4, HW // PBB),
        in_specs=[
            pl.BlockSpec((1, 1, 8, PBB), lambda b, g, c, j: (b, g, 0, j)),
            pl.BlockSpec((1, 1, 1, PBB), lambda b, g, c, j: (b, c, 0, j)),
        ],
        out_specs=pl.BlockSpec((1, 1, 1, 8, PBB),
                               lambda b, g, c, j: (b, g, c, 0, j)),
    )(inp12, wgt.reshape(B, 4, 1, HW))


def _sc_body(v, wgt, idx, zeros, out,
             a0, a1, a2, a3, a4, a5, a6, a7, idxb, vb):
    accs = (a0, a1, a2, a3, a4, a5, a6, a7)
    b = lax.axis_index("c")
    s = lax.axis_index("s")
    base = s * PT

    for c in range(4):
        pltpu.sync_copy(idx.at[b, c, pl.ds(base, PT)],
                        idxb.at[pl.ds(c * PT, PT)])
    for ch in range(8):
        pltpu.sync_copy(zeros, accs[ch].at[pl.ds(base, PT)])
    plsc.subcore_barrier()

    def per_group(g, carry):
        for c in range(4):
            for ch in range(8):
                pltpu.sync_copy(v.at[b, g, c, ch, pl.ds(base, PT)], vb)
                pltpu.sync_copy(vb, accs[ch].at[idxb.at[pl.ds(c * PT, PT)]],
                                add=True)
        plsc.subcore_barrier()
        for ch in range(8):
            pltpu.sync_copy(accs[ch].at[pl.ds(base, PT)],
                            out.at[b, g, ch, pl.ds(base, PT)])
        for ch in range(8):
            pltpu.sync_copy(zeros, accs[ch].at[pl.ds(base, PT)])
        plsc.subcore_barrier()
        return carry

    lax.fori_loop(0, NG, per_group, 0)

    for c in range(4):
        pltpu.sync_copy(wgt.at[b, c, pl.ds(base, PT)], vb)
        pltpu.sync_copy(vb, a0.at[idxb.at[pl.ds(c * PT, PT)]], add=True)
    plsc.subcore_barrier()
    pltpu.sync_copy(a0.at[pl.ds(base, PT)],
                    out.at[b, NG, 0, pl.ds(base, PT)])
    plsc.subcore_barrier()


@jax.jit
def _sc_splat(v, wgt, idx, zeros):
    mesh = plsc.VectorSubcoreMesh(
        core_axis_name="c", subcore_axis_name="s", num_cores=2, num_subcores=NS)
    f = pl.kernel(
        _sc_body,
        out_type=jax.ShapeDtypeStruct((B, NG + 1, 8, HW), jnp.float32),
        mesh=mesh,
        scratch_types=(
            [pltpu.VMEM_SHARED((HW,), jnp.float32) for _ in range(8)]
            + [pltpu.VMEM((4 * PT,), jnp.int32),
               pltpu.VMEM((PT,), jnp.float32)]
        ),
    )
    return f(v, wgt, idx, zeros)


def _normalize_body(num_ref, den_ref, out_ref):
    out_ref[0, 0] = num_ref[0, 0] / (den_ref[0, 0, 0:1, :] + 0.0000001)


def _normalize(splat):
    return pl.pallas_call(
        _normalize_body,
        out_shape=jax.ShapeDtypeStruct((B, NG, 8, HW), jnp.float32),
        grid=(B, NG, HW // PBB),
        in_specs=[
            pl.BlockSpec((1, 1, 8, PBB), lambda b, g, j: (b, g, 0, j)),
            pl.BlockSpec((1, 1, 8, PBB), lambda b, g, j: (b, NG, 0, j)),
        ],
        out_specs=pl.BlockSpec((1, 1, 8, PBB), lambda b, g, j: (b, g, 0, j)),
    )(splat, splat)


def kernel(tenInput, tenFlow, tenMetric):
    inp12 = tenInput.reshape(B, NG, 8, HW)
    flow = tenFlow.reshape(B, 2, HW)
    met = tenMetric.reshape(B, 1, HW)
    zeros = jnp.zeros((PT,), jnp.float32)
    wgt, idx = _corners(flow, met)
    v = _values(inp12, wgt)
    splat = _sc_splat(v, wgt, idx, zeros)
    return _normalize(splat).reshape(B, C, H, W)


# speedup vs baseline: 2.3049x; 1.1437x over previous
import jax
import jax.numpy as jnp
from jax import lax
from jax.experimental import pallas as pl
from jax.experimental.pallas import tpu as pltpu
from jax.experimental.pallas import tpu_sc as plsc

H = 384
W = 384
HW = H * W
B = 2
C = 96
NG = 12
NS = 16
PT = HW // NS
PBA = 2048
PBB = 16384

_CORNERS = ((0, 0), (1, 0), (0, 1), (1, 1))


def _corners_body(flow_ref, met_ref, wgt_ref, idx_ref):
    j = pl.program_id(1)
    f0 = flow_ref[0, 0:1, :]
    f1 = flow_ref[0, 1:2, :]
    e = jnp.exp(met_ref[0, :, :])
    pix = j * PBA + lax.broadcasted_iota(jnp.int32, (1, PBA), 1)
    y = pix // W
    x = pix - y * W
    ox = jnp.clip(x.astype(jnp.float32) + f0, -3.0, W + 2.0)
    oy = jnp.clip(y.astype(jnp.float32) + f1, -3.0, H + 2.0)
    nwx = jnp.floor(ox)
    nwy = jnp.floor(oy)
    for ci, (dx, dy) in enumerate(_CORNERS):
        cx = nwx + float(dx)
        cy = nwy + float(dy)
        wx = ((nwx + 1.0) - ox) if dx == 0 else (ox - nwx)
        wy = ((nwy + 1.0) - oy) if dy == 0 else (oy - nwy)
        valid = ((cx >= 0.0) & (cx <= W - 1.0) & (cy >= 0.0) & (cy <= H - 1.0))
        w = jnp.where(valid, wx * wy, 0.0) * e
        cxi = jnp.clip(cx.astype(jnp.int32), 0, W - 1)
        cyi = jnp.clip(cy.astype(jnp.int32), 0, H - 1)
        wgt_ref[0, ci:ci + 1, :] = w
        idx_ref[0, ci:ci + 1, :] = cyi * W + cxi


def _corners(flow, met):
    return pl.pallas_call(
        _corners_body,
        out_shape=(jax.ShapeDtypeStruct((B, 4, HW), jnp.float32),
                   jax.ShapeDtypeStruct((B, 4, HW), jnp.int32)),
        grid=(B, HW // PBA),
        in_specs=[
            pl.BlockSpec((1, 2, PBA), lambda b, j: (b, 0, j)),
            pl.BlockSpec((1, 1, PBA), lambda b, j: (b, 0, j)),
        ],
        out_specs=(pl.BlockSpec((1, 4, PBA), lambda b, j: (b, 0, j)),
                   pl.BlockSpec((1, 4, PBA), lambda b, j: (b, 0, j))),
    )(flow, met)


def _values_body(inp_ref, wgt_ref, v_ref):
    v_ref[0, 0, 0] = inp_ref[0, 0] * wgt_ref[0, 0]


def _values(inp12, wgt):
    return pl.pallas_call(
        _values_body,
        out_shape=jax.ShapeDtypeStruct((B, NG, 4, 8, HW), jnp.float32),
        grid=(B, NG, 4, HW // PBB),
        in_specs=[
            pl.BlockSpec((1, 1, 8, PBB), lambda b, g, c, j: (b, g, 0, j)),
            pl.BlockSpec((1, 1, 1, PBB), lambda b, g, c, j: (b, c, 0, j)),
        ],
        out_specs=pl.BlockSpec((1, 1, 1, 8, PBB),
                               lambda b, g, c, j: (b, g, c, 0, j)),
    )(inp12, wgt.reshape(B, 4, 1, HW))


def _sc_body(v, wgt, idx, zeros, out,
             a0, a1, a2, a3, a4, a5, a6, a7, idxb, vb0, vb1,
             sl0, sl1, sa0, sa1):
    accs = (a0, a1, a2, a3, a4, a5, a6, a7)
    vbs = (vb0, vb1)
    sls = (sl0, sl1)
    sas = (sa0, sa1)
    b = lax.axis_index("c")
    s = lax.axis_index("s")
    base = s * PT

    for c in range(4):
        pltpu.sync_copy(idx.at[b, c, pl.ds(base, PT)],
                        idxb.at[pl.ds(c * PT, PT)])
    for ch in range(8):
        pltpu.sync_copy(zeros, accs[ch].at[pl.ds(base, PT)])
    plsc.subcore_barrier()

    def src_of(g, i):
        c = i // 8
        ch = i - c * 8
        return v.at[b, g, c, ch, pl.ds(base, PT)]

    def add_of(i, slot):
        c = i // 8
        ch = i - c * 8
        return pltpu.make_async_copy(
            vbs[slot], accs[ch].at[idxb.at[pl.ds(c * PT, PT)]], sas[slot])

    def per_group(g, carry):
        pltpu.make_async_copy(src_of(g, 0), vbs[0], sls[0]).start()
        for i in range(32):
            slot = i & 1
            pltpu.make_async_copy(src_of(g, i), vbs[slot], sls[slot]).wait()
            add_of(i, slot).start(add=True)
            if i + 1 < 32:
                other = 1 - slot
                if i >= 1:
                    add_of(i - 1, other).wait()
                pltpu.make_async_copy(
                    src_of(g, i + 1), vbs[other], sls[other]).start()
        add_of(30, 0).wait()
        add_of(31, 1).wait()
        plsc.subcore_barrier()
        for ch in range(8):
            pltpu.sync_copy(accs[ch].at[pl.ds(base, PT)],
                            out.at[b, g, ch, pl.ds(base, PT)])
        for ch in range(8):
            pltpu.sync_copy(zeros, accs[ch].at[pl.ds(base, PT)])
        plsc.subcore_barrier()
        return carry

    lax.fori_loop(0, NG, per_group, 0)

    for c in range(4):
        pltpu.sync_copy(wgt.at[b, c, pl.ds(base, PT)], vb0)
        pltpu.sync_copy(vb0, a0.at[idxb.at[pl.ds(c * PT, PT)]], add=True)
    plsc.subcore_barrier()
    pltpu.sync_copy(a0.at[pl.ds(base, PT)],
                    out.at[b, NG, 0, pl.ds(base, PT)])
    plsc.subcore_barrier()


@jax.jit
def _sc_splat(v, wgt, idx, zeros):
    mesh = plsc.VectorSubcoreMesh(
        core_axis_name="c", subcore_axis_name="s", num_cores=2, num_subcores=NS)
    f = pl.kernel(
        _sc_body,
        out_type=jax.ShapeDtypeStruct((B, NG + 1, 8, HW), jnp.float32),
        mesh=mesh,
        scratch_types=(
            [pltpu.VMEM_SHARED((HW,), jnp.float32) for _ in range(8)]
            + [pltpu.VMEM((4 * PT,), jnp.int32),
               pltpu.VMEM((PT,), jnp.float32),
               pltpu.VMEM((PT,), jnp.float32)]
            + [pltpu.SemaphoreType.DMA] * 4
        ),
    )
    return f(v, wgt, idx, zeros)


def _normalize_body(num_ref, den_ref, out_ref):
    out_ref[0, 0] = num_ref[0, 0] / (den_ref[0, 0, 0:1, :] + 0.0000001)


def _normalize(splat):
    return pl.pallas_call(
        _normalize_body,
        out_shape=jax.ShapeDtypeStruct((B, NG, 8, HW), jnp.float32),
        grid=(B, NG, HW // PBB),
        in_specs=[
            pl.BlockSpec((1, 1, 8, PBB), lambda b, g, j: (b, g, 0, j)),
            pl.BlockSpec((1, 1, 8, PBB), lambda b, g, j: (b, NG, 0, j)),
        ],
        out_specs=pl.BlockSpec((1, 1, 8, PBB), lambda b, g, j: (b, g, 0, j)),
    )(splat, splat)


def kernel(tenInput, tenFlow, tenMetric):
    inp12 = tenInput.reshape(B, NG, 8, HW)
    flow = tenFlow.reshape(B, 2, HW)
    met = tenMetric.reshape(B, 1, HW)
    zeros = jnp.zeros((PT,), jnp.float32)
    wgt, idx = _corners(flow, met)
    v = _values(inp12, wgt)
    splat = _sc_splat(v, wgt, idx, zeros)
    return _normalize(splat).reshape(B, C, H, W)


# speedup vs baseline: 2.7027x; 1.1726x over previous
import jax
import jax.numpy as jnp
from jax import lax
from jax.experimental import pallas as pl
from jax.experimental.pallas import tpu as pltpu
from jax.experimental.pallas import tpu_sc as plsc

H = 384
W = 384
HW = H * W
B = 2
C = 96
NG = 12
NS = 16
PT = HW // NS
PBA = 2048
PBB = 49152

_CORNERS = ((0, 0), (1, 0), (0, 1), (1, 1))


def _corners_body(flow_ref, met_ref, wgt_ref, idx_ref):
    j = pl.program_id(1)
    f0 = flow_ref[0, 0:1, :]
    f1 = flow_ref[0, 1:2, :]
    e = jnp.exp(met_ref[0, :, :])
    pix = j * PBA + lax.broadcasted_iota(jnp.int32, (1, PBA), 1)
    y = pix // W
    x = pix - y * W
    ox = jnp.clip(x.astype(jnp.float32) + f0, -3.0, W + 2.0)
    oy = jnp.clip(y.astype(jnp.float32) + f1, -3.0, H + 2.0)
    nwx = jnp.floor(ox)
    nwy = jnp.floor(oy)
    for ci, (dx, dy) in enumerate(_CORNERS):
        cx = nwx + float(dx)
        cy = nwy + float(dy)
        wx = ((nwx + 1.0) - ox) if dx == 0 else (ox - nwx)
        wy = ((nwy + 1.0) - oy) if dy == 0 else (oy - nwy)
        valid = ((cx >= 0.0) & (cx <= W - 1.0) & (cy >= 0.0) & (cy <= H - 1.0))
        w = jnp.where(valid, wx * wy, 0.0) * e
        cxi = jnp.clip(cx.astype(jnp.int32), 0, W - 1)
        cyi = jnp.clip(cy.astype(jnp.int32), 0, H - 1)
        wgt_ref[0, ci:ci + 1, :] = w
        idx_ref[0, ci:ci + 1, :] = cyi * W + cxi


def _corners(flow, met):
    return pl.pallas_call(
        _corners_body,
        out_shape=(jax.ShapeDtypeStruct((B, 4, HW), jnp.float32),
                   jax.ShapeDtypeStruct((B, 4, HW), jnp.int32)),
        grid=(B, HW // PBA),
        in_specs=[
            pl.BlockSpec((1, 2, PBA), lambda b, j: (b, 0, j)),
            pl.BlockSpec((1, 1, PBA), lambda b, j: (b, 0, j)),
        ],
        out_specs=(pl.BlockSpec((1, 4, PBA), lambda b, j: (b, 0, j)),
                   pl.BlockSpec((1, 4, PBA), lambda b, j: (b, 0, j))),
    )(flow, met)


def _values_body(inp_ref, wgt_ref, v_ref):
    v_ref[0, 0, 0] = inp_ref[0, 0] * wgt_ref[0, 0]


def _values(inp12, wgt):
    return pl.pallas_call(
        _values_body,
        out_shape=jax.ShapeDtypeStruct((B, NG, 4, 8, HW), jnp.float32),
        grid=(B, NG, 4, HW // PBB),
        in_specs=[
            pl.BlockSpec((1, 1, 8, PBB), lambda b, g, c, j: (b, g, 0, j)),
            pl.BlockSpec((1, 1, 1, PBB), lambda b, g, c, j: (b, c, 0, j)),
        ],
        out_specs=pl.BlockSpec((1, 1, 1, 8, PBB),
                               lambda b, g, c, j: (b, g, c, 0, j)),
    )(inp12, wgt.reshape(B, 4, 1, HW))


def _sc_body(v, wgt, idx, zeros, out,
             a0, a1, a2, a3, a4, a5, a6, a7, idxb, vb0, vb1,
             sl0, sl1, sa0, sa1):
    accs = (a0, a1, a2, a3, a4, a5, a6, a7)
    vbs = (vb0, vb1)
    sls = (sl0, sl1)
    sas = (sa0, sa1)
    b = lax.axis_index("c")
    s = lax.axis_index("s")
    base = s * PT

    for c in range(4):
        pltpu.sync_copy(idx.at[b, c, pl.ds(base, PT)],
                        idxb.at[pl.ds(c * PT, PT)])
    for ch in range(8):
        pltpu.sync_copy(zeros, accs[ch].at[pl.ds(base, PT)])
    plsc.subcore_barrier()

    def src_of(g, i):
        c = i // 8
        ch = i - c * 8
        return v.at[b, g, c, ch, pl.ds(base, PT)]

    def add_of(i, slot):
        c = i // 8
        ch = i - c * 8
        return pltpu.make_async_copy(
            vbs[slot], accs[ch].at[idxb.at[pl.ds(c * PT, PT)]], sas[slot])

    def per_group(g, carry):
        pltpu.make_async_copy(src_of(g, 0), vbs[0], sls[0]).start()
        for i in range(32):
            slot = i & 1
            pltpu.make_async_copy(src_of(g, i), vbs[slot], sls[slot]).wait()
            add_of(i, slot).start(add=True)
            if i + 1 < 32:
                other = 1 - slot
                if i >= 1:
                    add_of(i - 1, other).wait()
                pltpu.make_async_copy(
                    src_of(g, i + 1), vbs[other], sls[other]).start()
        add_of(30, 0).wait()
        add_of(31, 1).wait()
        plsc.subcore_barrier()
        drains = [pltpu.make_async_copy(accs[ch].at[pl.ds(base, PT)],
                                        out.at[b, g, ch, pl.ds(base, PT)],
                                        sl0)
                  for ch in range(8)]
        for d in drains:
            d.start()
        for d in drains:
            d.wait()
        zs = [pltpu.make_async_copy(zeros, accs[ch].at[pl.ds(base, PT)], sl1)
              for ch in range(8)]
        for z in zs:
            z.start()
        for z in zs:
            z.wait()
        plsc.subcore_barrier()
        return carry

    lax.fori_loop(0, NG, per_group, 0)

    for c in range(4):
        pltpu.sync_copy(wgt.at[b, c, pl.ds(base, PT)], vb0)
        pltpu.sync_copy(vb0, a0.at[idxb.at[pl.ds(c * PT, PT)]], add=True)
    plsc.subcore_barrier()
    pltpu.sync_copy(a0.at[pl.ds(base, PT)],
                    out.at[b, NG, 0, pl.ds(base, PT)])
    plsc.subcore_barrier()


@jax.jit
def _sc_splat(v, wgt, idx, zeros):
    mesh = plsc.VectorSubcoreMesh(
        core_axis_name="c", subcore_axis_name="s", num_cores=2, num_subcores=NS)
    f = pl.kernel(
        _sc_body,
        out_type=jax.ShapeDtypeStruct((B, NG + 1, 8, HW), jnp.float32),
        mesh=mesh,
        scratch_types=(
            [pltpu.VMEM_SHARED((HW,), jnp.float32) for _ in range(8)]
            + [pltpu.VMEM((4 * PT,), jnp.int32),
               pltpu.VMEM((PT,), jnp.float32),
               pltpu.VMEM((PT,), jnp.float32)]
            + [pltpu.SemaphoreType.DMA] * 4
        ),
    )
    return f(v, wgt, idx, zeros)


def _normalize_body(num_ref, den_ref, out_ref):
    out_ref[0, 0] = num_ref[0, 0] / (den_ref[0, 0, 0:1, :] + 0.0000001)


def _normalize(splat):
    return pl.pallas_call(
        _normalize_body,
        out_shape=jax.ShapeDtypeStruct((B, NG, 8, HW), jnp.float32),
        grid=(B, NG, HW // PBB),
        in_specs=[
            pl.BlockSpec((1, 1, 8, PBB), lambda b, g, j: (b, g, 0, j)),
            pl.BlockSpec((1, 1, 8, PBB), lambda b, g, j: (b, NG, 0, j)),
        ],
        out_specs=pl.BlockSpec((1, 1, 8, PBB), lambda b, g, j: (b, g, 0, j)),
    )(splat, splat)


def kernel(tenInput, tenFlow, tenMetric):
    inp12 = tenInput.reshape(B, NG, 8, HW)
    flow = tenFlow.reshape(B, 2, HW)
    met = tenMetric.reshape(B, 1, HW)
    zeros = jnp.zeros((PT,), jnp.float32)
    wgt, idx = _corners(flow, met)
    v = _values(inp12, wgt)
    splat = _sc_splat(v, wgt, idx, zeros)
    return _normalize(splat).reshape(B, C, H, W)


# speedup vs baseline: 2.7749x; 1.0267x over previous
import jax
import jax.numpy as jnp
from jax import lax
from jax.experimental import pallas as pl
from jax.experimental.pallas import tpu as pltpu
from jax.experimental.pallas import tpu_sc as plsc

H = 384
W = 384
HW = H * W
B = 2
C = 96
NG = 12
NS = 16
PT = HW // NS
PBA = 2048
PBB = 49152

_CORNERS = ((0, 0), (1, 0), (0, 1), (1, 1))


def _corners_body(flow_ref, met_ref, wgt_ref, idx_ref):
    j = pl.program_id(1)
    f0 = flow_ref[0, 0:1, :]
    f1 = flow_ref[0, 1:2, :]
    e = jnp.exp(met_ref[0, :, :])
    pix = j * PBA + lax.broadcasted_iota(jnp.int32, (1, PBA), 1)
    y = pix // W
    x = pix - y * W
    ox = jnp.clip(x.astype(jnp.float32) + f0, -3.0, W + 2.0)
    oy = jnp.clip(y.astype(jnp.float32) + f1, -3.0, H + 2.0)
    nwx = jnp.floor(ox)
    nwy = jnp.floor(oy)
    for ci, (dx, dy) in enumerate(_CORNERS):
        cx = nwx + float(dx)
        cy = nwy + float(dy)
        wx = ((nwx + 1.0) - ox) if dx == 0 else (ox - nwx)
        wy = ((nwy + 1.0) - oy) if dy == 0 else (oy - nwy)
        valid = ((cx >= 0.0) & (cx <= W - 1.0) & (cy >= 0.0) & (cy <= H - 1.0))
        w = jnp.where(valid, wx * wy, 0.0) * e
        cxi = jnp.clip(cx.astype(jnp.int32), 0, W - 1)
        cyi = jnp.clip(cy.astype(jnp.int32), 0, H - 1)
        wgt_ref[0, ci:ci + 1, :] = w
        idx_ref[0, ci:ci + 1, :] = cyi * W + cxi


def _corners(flow, met):
    return pl.pallas_call(
        _corners_body,
        out_shape=(jax.ShapeDtypeStruct((B, 4, HW), jnp.float32),
                   jax.ShapeDtypeStruct((B, 4, HW), jnp.int32)),
        grid=(B, HW // PBA),
        in_specs=[
            pl.BlockSpec((1, 2, PBA), lambda b, j: (b, 0, j)),
            pl.BlockSpec((1, 1, PBA), lambda b, j: (b, 0, j)),
        ],
        out_specs=(pl.BlockSpec((1, 4, PBA), lambda b, j: (b, 0, j)),
                   pl.BlockSpec((1, 4, PBA), lambda b, j: (b, 0, j))),
    )(flow, met)


def _values_body(inp_ref, wgt_ref, v_ref):
    v_ref[0, 0, 0] = inp_ref[0, 0] * wgt_ref[0, 0]


def _values(inp12, wgt):
    return pl.pallas_call(
        _values_body,
        out_shape=jax.ShapeDtypeStruct((B, NG, 4, 8, HW), jnp.float32),
        grid=(B, NG, 4, HW // PBB),
        in_specs=[
            pl.BlockSpec((1, 1, 8, PBB), lambda b, g, c, j: (b, g, 0, j)),
            pl.BlockSpec((1, 1, 1, PBB), lambda b, g, c, j: (b, c, 0, j)),
        ],
        out_specs=pl.BlockSpec((1, 1, 1, 8, PBB),
                               lambda b, g, c, j: (b, g, c, 0, j)),
    )(inp12, wgt.reshape(B, 4, 1, HW))


def _sc_body(v, wgt, idx, zeros, out,
             a0, a1, a2, a3, a4, a5, a6, a7, idxb, vb0, vb1,
             sl0, sl1, sa0, sa1):
    accs = (a0, a1, a2, a3, a4, a5, a6, a7)
    vbs = (vb0, vb1)
    sls = (sl0, sl1)
    sas = (sa0, sa1)
    b = lax.axis_index("c")
    s = lax.axis_index("s")
    base = s * PT

    for c in range(4):
        pltpu.sync_copy(idx.at[b, c, pl.ds(base, PT)],
                        idxb.at[pl.ds(c * PT, PT)])
    for ch in range(8):
        pltpu.sync_copy(zeros, accs[ch].at[pl.ds(base, PT)])
    plsc.subcore_barrier()

    def src_of(g, i):
        c = i // 8
        ch = i - c * 8
        return v.at[b, g, c, ch, pl.ds(base, PT)]

    def add_of(i, slot):
        c = i // 8
        ch = i - c * 8
        return pltpu.make_async_copy(
            vbs[slot], accs[ch].at[idxb.at[pl.ds(c * PT, PT)]], sas[slot])

    def per_group(g, carry):
        for i in range(32):
            slot = i & 1
            pltpu.make_async_copy(src_of(g, i), vbs[slot], sls[slot]).wait()
            add_of(i, slot).start(add=True)
            if i + 1 < 32:
                other = 1 - slot
                if i >= 1:
                    add_of(i - 1, other).wait()
                pltpu.make_async_copy(
                    src_of(g, i + 1), vbs[other], sls[other]).start()
        add_of(30, 0).wait()
        add_of(31, 1).wait()
        gg = jnp.minimum(g + 1, NG - 1)
        pltpu.make_async_copy(
            v.at[b, gg, 0, 0, pl.ds(base, PT)], vbs[0], sls[0]).start()
        plsc.subcore_barrier()
        drains = [pltpu.make_async_copy(accs[ch].at[pl.ds(base, PT)],
                                        out.at[b, g, ch, pl.ds(base, PT)],
                                        sa0)
                  for ch in range(8)]
        zs = [pltpu.make_async_copy(zeros, accs[ch].at[pl.ds(base, PT)], sa1)
              for ch in range(8)]
        for d in drains:
            d.start()
        for ch in range(8):
            drains[ch].wait()
            zs[ch].start()
        for z in zs:
            z.wait()
        plsc.subcore_barrier()
        return carry

    pltpu.make_async_copy(src_of(0, 0), vbs[0], sls[0]).start()
    lax.fori_loop(0, NG, per_group, 0)
    pltpu.make_async_copy(
        v.at[b, NG - 1, 0, 0, pl.ds(base, PT)], vbs[0], sls[0]).wait()

    for c in range(4):
        pltpu.sync_copy(wgt.at[b, c, pl.ds(base, PT)], vb0)
        pltpu.sync_copy(vb0, a0.at[idxb.at[pl.ds(c * PT, PT)]], add=True)
    plsc.subcore_barrier()
    pltpu.sync_copy(a0.at[pl.ds(base, PT)],
                    out.at[b, NG, 0, pl.ds(base, PT)])
    plsc.subcore_barrier()


@jax.jit
def _sc_splat(v, wgt, idx, zeros):
    mesh = plsc.VectorSubcoreMesh(
        core_axis_name="c", subcore_axis_name="s", num_cores=2, num_subcores=NS)
    f = pl.kernel(
        _sc_body,
        out_type=jax.ShapeDtypeStruct((B, NG + 1, 8, HW), jnp.float32),
        mesh=mesh,
        scratch_types=(
            [pltpu.VMEM_SHARED((HW,), jnp.float32) for _ in range(8)]
            + [pltpu.VMEM((4 * PT,), jnp.int32),
               pltpu.VMEM((PT,), jnp.float32),
               pltpu.VMEM((PT,), jnp.float32)]
            + [pltpu.SemaphoreType.DMA] * 4
        ),
    )
    return f(v, wgt, idx, zeros)


def _normalize_body(num_ref, den_ref, out_ref):
    out_ref[0, 0] = num_ref[0, 0] / (den_ref[0, 0, 0:1, :] + 0.0000001)


def _normalize(splat):
    return pl.pallas_call(
        _normalize_body,
        out_shape=jax.ShapeDtypeStruct((B, NG, 8, HW), jnp.float32),
        grid=(B, NG, HW // PBB),
        in_specs=[
            pl.BlockSpec((1, 1, 8, PBB), lambda b, g, j: (b, g, 0, j)),
            pl.BlockSpec((1, 1, 8, PBB), lambda b, g, j: (b, NG, 0, j)),
        ],
        out_specs=pl.BlockSpec((1, 1, 8, PBB), lambda b, g, j: (b, g, 0, j)),
    )(splat, splat)


def kernel(tenInput, tenFlow, tenMetric):
    inp12 = tenInput.reshape(B, NG, 8, HW)
    flow = tenFlow.reshape(B, 2, HW)
    met = tenMetric.reshape(B, 1, HW)
    zeros = jnp.zeros((PT,), jnp.float32)
    wgt, idx = _corners(flow, met)
    v = _values(inp12, wgt)
    splat = _sc_splat(v, wgt, idx, zeros)
    return _normalize(splat).reshape(B, C, H, W)
